# baseline (speedup 1.0000x reference)
import functools

import jax
import jax.numpy as jnp
from jax import lax
from jax.experimental import pallas as pl
from jax.experimental.pallas import tpu as pltpu
from jax.experimental.pallas import tpu_sc as plsc

_F32 = jnp.float32
_NT = 32
_NSUB = 16


def _sc_mesh():
    return plsc.VectorSubcoreMesh(core_axis_name="c", subcore_axis_name="s")


_SC_PARAMS = dict(
    compiler_params=pltpu.CompilerParams(
        needs_layout_passes=False, use_tc_tiling_on_sc=False),
)


@functools.lru_cache(maxsize=None)
def _build_deg_kernel(E, NP):
    EPT = E // _NT

    def body(ei_hbm, out_hbm, dstb, acc):
        cid = lax.axis_index("c")
        sid = lax.axis_index("s")
        t = cid * _NSUB + sid
        zero = jnp.zeros((16,), _F32)

        def zb(i, c):
            acc[pl.ds(i * 16, 16)] = zero
            return c

        lax.fori_loop(0, NP // 16, zb, 0)
        pltpu.sync_copy(ei_hbm.at[1, pl.ds(t * EPT, EPT)], dstb)
        ones = jnp.ones((16,), _F32)

        def eb(i, c):
            base = i * 80
            for u in range(5):
                idx = dstb[pl.ds(base + u * 16, 16)]
                plsc.addupdate_scatter(acc, [idx], ones)
            return c

        lax.fori_loop(0, EPT // 80, eb, 0)
        pltpu.sync_copy(acc, out_hbm.at[t])

    return pl.kernel(
        body,
        out_type=jax.ShapeDtypeStruct((_NT, NP), _F32),
        mesh=_sc_mesh(),
        scratch_types=[
            pltpu.VMEM((EPT,), jnp.int32),
            pltpu.VMEM((NP,), _F32),
        ],
        **_SC_PARAMS,
    )


def _tc_prep(p, x, W1, NP):
    N = x.shape[0]
    DH = W1.shape[1]

    def body(p_ref, x_ref, w1_ref, g1_ref, dis2_ref, dis1_ref):
        deg = jnp.sum(p_ref[...], axis=0) + 1.0
        dis = lax.rsqrt(deg)
        dis1_ref[...] = dis
        dis2_ref[...] = dis[:, None]
        h1 = jnp.dot(x_ref[...], w1_ref[...], preferred_element_type=_F32)
        g1_ref[pl.ds(0, N), :] = h1 * dis[:N][:, None]

    return pl.pallas_call(
        body,
        out_shape=[
            jax.ShapeDtypeStruct((NP, DH), _F32),
            jax.ShapeDtypeStruct((NP, 1), _F32),
            jax.ShapeDtypeStruct((NP,), _F32),
        ],
    )(p, x, W1)


@functools.lru_cache(maxsize=None)
def _build_prop1_kernel(E, NP, DH, CH, NSLOT):
    EPT = E // _NT
    NIT = EPT // CH
    NPT = NP // _NSUB

    def body(ei_hbm, g1_hbm, out_hbm, srcb, dstb, rows, zbuf, sacc, *sems):
        gsem = sems[:NSLOT]
        ssem = sems[NSLOT:]
        cid = lax.axis_index("c")
        sid = lax.axis_index("s")
        t = cid * _NSUB + sid
        pltpu.sync_copy(ei_hbm.at[0, pl.ds(t * EPT, EPT)], srcb)
        pltpu.sync_copy(ei_hbm.at[1, pl.ds(t * EPT, EPT)], dstb)
        zero = jnp.zeros((16,), _F32)

        def zb(i, c):
            def zc(j, c2):
                zbuf[i, pl.ds(j * 16, 16)] = zero
                return c2
            lax.fori_loop(0, DH // 16, zc, 0)
            return c

        lax.fori_loop(0, 128, zb, 0)
        for r in range(NPT // 128):
            pltpu.sync_copy(zbuf, sacc.at[pl.ds(sid * NPT + r * 128, 128)])
        plsc.subcore_barrier()

        def start_g(i, s):
            pltpu.async_copy(g1_hbm.at[srcb.at[pl.ds(i * CH, CH)]],
                             rows.at[s], gsem[s])

        def wait_g(s):
            pltpu.make_async_copy(g1_hbm.at[srcb.at[pl.ds(0, CH)]],
                                  rows.at[s], gsem[s]).wait()

        def start_s(i, s):
            pltpu.async_copy(rows.at[s],
                             sacc.at[dstb.at[pl.ds(i * CH, CH)]],
                             ssem[s], add=True)

        def wait_s(s):
            pltpu.make_async_copy(rows.at[s],
                                  sacc.at[dstb.at[pl.ds(0, CH)]],
                                  ssem[s]).wait()

        for s in range(NSLOT):
            start_g(s, s)

        def eb(j, c):
            i0 = NSLOT * j
            for s in range(NSLOT):
                wait_g(s)
                start_s(i0 + s, s)

            @pl.when(j < NIT // NSLOT - 1)
            def _():
                for s in range(NSLOT):
                    wait_s(s)
                    start_g(i0 + NSLOT + s, s)

            return c

        lax.fori_loop(0, NIT // NSLOT, eb, 0)
        for s in range(NSLOT):
            wait_s(s)
        plsc.subcore_barrier()
        pltpu.sync_copy(sacc.at[pl.ds(sid * NPT, NPT)],
                        out_hbm.at[cid, pl.ds(sid * NPT, NPT)])

    return pl.kernel(
        body,
        out_type=jax.ShapeDtypeStruct((2, NP, DH), _F32),
        mesh=_sc_mesh(),
        scratch_types=(
            [pltpu.VMEM((EPT,), jnp.int32),
             pltpu.VMEM((EPT,), jnp.int32),
             pltpu.VMEM((NSLOT, CH, DH), _F32),
             pltpu.VMEM((128, DH), _F32),
             pltpu.VMEM_SHARED((NP, DH), _F32)]
            + [pltpu.SemaphoreType.DMA] * (2 * NSLOT)
        ),
        **_SC_PARAMS,
    )


def _tc_mid(accp, g1, dis2d, dis1, b1, W2, b2):
    NP, DH = g1.shape

    def body(a_ref, g1_ref, dis2_ref, dis1_ref, b1_ref, w2_ref, b2_ref,
             g2_ref, c2_ref):
        acc = a_ref[0] + a_ref[1] + g1_ref[...]
        out1 = acc * dis2_ref[...] + b1_ref[...]
        h = jnp.maximum(out1, 0.0)
        h2 = jnp.sum(h * w2_ref[...], axis=1)
        dis = dis1_ref[...]
        g2 = dis * h2
        g2_ref[...] = g2
        c2_ref[...] = dis * g2 + b2_ref[...]

    return pl.pallas_call(
        body,
        out_shape=[
            jax.ShapeDtypeStruct((NP,), _F32),
            jax.ShapeDtypeStruct((NP,), _F32),
        ],
    )(accp, g1, dis2d, dis1, b1.reshape(1, DH), W2.reshape(1, DH),
      b2.reshape(1))


@functools.lru_cache(maxsize=None)
def _build_prop2_kernel(E, NP):
    EPT = E // _NT
    NPT = NP // _NSUB

    def body(g2_hbm, ei_hbm, out_hbm, g2b, srcb, dstb, acc2, outb, redb,
             slots):
        cid = lax.axis_index("c")
        sid = lax.axis_index("s")
        t = cid * _NSUB + sid
        pltpu.sync_copy(g2_hbm, g2b)
        zero = jnp.zeros((16,), _F32)

        def zb(i, c):
            base = i * 64
            for u in range(4):
                acc2[pl.ds(base + u * 16, 16)] = zero
            return c

        lax.fori_loop(0, NP // 64, zb, 0)
        pltpu.sync_copy(ei_hbm.at[0, pl.ds(t * EPT, EPT)], srcb)
        pltpu.sync_copy(ei_hbm.at[1, pl.ds(t * EPT, EPT)], dstb)

        def eb(i, c):
            base = i * 80
            for u in range(5):
                s = srcb[pl.ds(base + u * 16, 16)]
                d = dstb[pl.ds(base + u * 16, 16)]
                v = plsc.load_gather(g2b, [s])
                plsc.addupdate_scatter(acc2, [d], v)
            return c

        lax.fori_loop(0, EPT // 80, eb, 0)
        pltpu.sync_copy(acc2, slots.at[sid])
        plsc.subcore_barrier()
        for j in range(_NSUB):
            pltpu.sync_copy(slots.at[j, pl.ds(sid * NPT, NPT)], redb.at[j])

        def cb(ci, c):
            base = ci * 16
            tot = redb[0, pl.ds(base, 16)]
            for j in range(1, _NSUB):
                tot = tot + redb[j, pl.ds(base, 16)]
            outb[pl.ds(base, 16)] = tot
            return c

        lax.fori_loop(0, NPT // 16, cb, 0)
        pltpu.sync_copy(outb, out_hbm.at[cid, pl.ds(sid * NPT, NPT)])

    return pl.kernel(
        body,
        out_type=jax.ShapeDtypeStruct((2, NP), _F32),
        mesh=_sc_mesh(),
        scratch_types=[
            pltpu.VMEM((NP,), _F32),
            pltpu.VMEM((EPT,), jnp.int32),
            pltpu.VMEM((EPT,), jnp.int32),
            pltpu.VMEM((NP,), _F32),
            pltpu.VMEM((NPT,), _F32),
            pltpu.VMEM((_NSUB, NPT), _F32),
            pltpu.VMEM_SHARED((_NSUB, NP), _F32),
        ],
        **_SC_PARAMS,
    )


def _tc_final(p2, dis1, c2):
    NP = dis1.shape[0]

    def body(p_ref, dis_ref, c2_ref, out_ref):
        out_ref[...] = (dis_ref[...] * (p_ref[0] + p_ref[1])
                        + c2_ref[...])

    return pl.pallas_call(
        body,
        out_shape=jax.ShapeDtypeStruct((NP,), _F32),
    )(p2, dis1, c2)


def kernel(x, edge_index, W1, b1, W2, b2):
    N, _ = x.shape
    E = edge_index.shape[1]
    DH = W1.shape[1]
    NP = ((N + 16 * 128 - 1) // (16 * 128)) * (16 * 128)
    CH = 40
    NSLOT = 10

    p = _build_deg_kernel(E, NP)(edge_index)
    g1, dis2d, dis = _tc_prep(p, x, W1, NP)
    accp = _build_prop1_kernel(E, NP, DH, CH, NSLOT)(edge_index, g1)
    g2, c2 = _tc_mid(accp, g1, dis2d, dis, b1, W2, b2)
    p2 = _build_prop2_kernel(E, NP)(g2, edge_index)
    out = _tc_final(p2, dis, c2)
    return out[:N, None]


# speedup vs baseline: 1.0259x; 1.0259x over previous
import functools

import jax
import jax.numpy as jnp
from jax import lax
from jax.experimental import pallas as pl
from jax.experimental.pallas import tpu as pltpu
from jax.experimental.pallas import tpu_sc as plsc

_F32 = jnp.float32
_NT = 32
_NSUB = 16


def _sc_mesh():
    return plsc.VectorSubcoreMesh(core_axis_name="c", subcore_axis_name="s")


_SC_PARAMS = dict(
    compiler_params=pltpu.CompilerParams(
        needs_layout_passes=False, use_tc_tiling_on_sc=False),
)


@functools.lru_cache(maxsize=None)
def _build_deg_kernel(E, NP):
    EPT = E // _NT

    def body(ei_hbm, out_hbm, dstb, acc):
        cid = lax.axis_index("c")
        sid = lax.axis_index("s")
        t = cid * _NSUB + sid
        zero = jnp.zeros((16,), _F32)

        def zb(i, c):
            acc[pl.ds(i * 16, 16)] = zero
            return c

        lax.fori_loop(0, NP // 16, zb, 0)
        pltpu.sync_copy(ei_hbm.at[1, pl.ds(t * EPT, EPT)], dstb)
        ones = jnp.ones((16,), _F32)

        def eb(i, c):
            base = i * 80
            for u in range(5):
                idx = dstb[pl.ds(base + u * 16, 16)]
                plsc.addupdate_scatter(acc, [idx], ones)
            return c

        lax.fori_loop(0, EPT // 80, eb, 0)
        pltpu.sync_copy(acc, out_hbm.at[t])

    return pl.kernel(
        body,
        out_type=jax.ShapeDtypeStruct((_NT, NP), _F32),
        mesh=_sc_mesh(),
        scratch_types=[
            pltpu.VMEM((EPT,), jnp.int32),
            pltpu.VMEM((NP,), _F32),
        ],
        **_SC_PARAMS,
    )


def _tc_prep(p, x, W1, NP):
    N = x.shape[0]
    DH = W1.shape[1]

    def body(p_ref, x_ref, w1_ref, g1_ref, dis2_ref, dis1_ref):
        deg = jnp.sum(p_ref[...], axis=0) + 1.0
        dis = lax.rsqrt(deg)
        dis1_ref[...] = dis
        dis2_ref[...] = dis[:, None]
        h1 = jnp.dot(x_ref[...], w1_ref[...], preferred_element_type=_F32)
        g1_ref[pl.ds(0, N), :] = h1 * dis[:N][:, None]

    return pl.pallas_call(
        body,
        out_shape=[
            jax.ShapeDtypeStruct((NP, DH), _F32),
            jax.ShapeDtypeStruct((NP, 1), _F32),
            jax.ShapeDtypeStruct((NP,), _F32),
        ],
    )(p, x, W1)


@functools.lru_cache(maxsize=None)
def _build_prop1_kernel(E, NP, DH, CH, NSLOT):
    EPT = E // _NT
    NIT = EPT // CH
    NPT = NP // _NSUB

    def body(ei_hbm, g1_hbm, out_hbm, srcb, dstb, rows, zbuf, sacc, *sems):
        gsem = sems[:NSLOT]
        ssem = sems[NSLOT:]
        cid = lax.axis_index("c")
        sid = lax.axis_index("s")
        t = cid * _NSUB + sid
        pltpu.sync_copy(ei_hbm.at[0, pl.ds(t * EPT, EPT)], srcb)
        pltpu.sync_copy(ei_hbm.at[1, pl.ds(t * EPT, EPT)], dstb)
        zero = jnp.zeros((16,), _F32)

        def zb(i, c):
            def zc(j, c2):
                zbuf[i, pl.ds(j * 16, 16)] = zero
                return c2
            lax.fori_loop(0, DH // 16, zc, 0)
            return c

        lax.fori_loop(0, 128, zb, 0)
        for r in range(NPT // 128):
            pltpu.sync_copy(zbuf, sacc.at[pl.ds(sid * NPT + r * 128, 128)])
        plsc.subcore_barrier()

        def start_g(i, s):
            pltpu.async_copy(g1_hbm.at[srcb.at[pl.ds(i * CH, CH)]],
                             rows.at[s], gsem[s])

        def wait_g(s):
            pltpu.make_async_copy(g1_hbm.at[srcb.at[pl.ds(0, CH)]],
                                  rows.at[s], gsem[s]).wait()

        def start_s(i, s):
            pltpu.async_copy(rows.at[s],
                             sacc.at[dstb.at[pl.ds(i * CH, CH)]],
                             ssem[s], add=True)

        def wait_s(s):
            pltpu.make_async_copy(rows.at[s],
                                  sacc.at[dstb.at[pl.ds(0, CH)]],
                                  ssem[s]).wait()

        for s in range(NSLOT):
            start_g(s, s)

        def eb(j, c):
            i0 = NSLOT * j
            for s in range(NSLOT):
                wait_g(s)
                start_s(i0 + s, s)

            @pl.when(j < NIT // NSLOT - 1)
            def _():
                for s in range(NSLOT):
                    wait_s(s)
                    start_g(i0 + NSLOT + s, s)

            return c

        lax.fori_loop(0, NIT // NSLOT, eb, 0)
        for s in range(NSLOT):
            wait_s(s)
        plsc.subcore_barrier()
        pltpu.sync_copy(sacc.at[pl.ds(sid * NPT, NPT)],
                        out_hbm.at[cid, pl.ds(sid * NPT, NPT)])

    return pl.kernel(
        body,
        out_type=jax.ShapeDtypeStruct((2, NP, DH), _F32),
        mesh=_sc_mesh(),
        scratch_types=(
            [pltpu.VMEM((EPT,), jnp.int32),
             pltpu.VMEM((EPT,), jnp.int32),
             pltpu.VMEM((NSLOT, CH, DH), _F32),
             pltpu.VMEM((128, DH), _F32),
             pltpu.VMEM_SHARED((NP, DH), _F32)]
            + [pltpu.SemaphoreType.DMA] * (2 * NSLOT)
        ),
        **_SC_PARAMS,
    )


def _tc_mid(accp, g1, dis2d, dis1, b1, W2, b2):
    NP, DH = g1.shape

    def body(a_ref, g1_ref, dis2_ref, dis1_ref, b1_ref, w2_ref, b2_ref,
             g2_ref, c2_ref):
        acc = a_ref[0] + a_ref[1] + g1_ref[...]
        out1 = acc * dis2_ref[...] + b1_ref[...]
        h = jnp.maximum(out1, 0.0)
        h2 = jnp.dot(h, w2_ref[...],
                     preferred_element_type=_F32)[:, 0]
        dis = dis1_ref[...]
        g2 = dis * h2
        g2_ref[...] = g2
        c2_ref[...] = dis * g2 + b2_ref[...]

    return pl.pallas_call(
        body,
        out_shape=[
            jax.ShapeDtypeStruct((NP,), _F32),
            jax.ShapeDtypeStruct((NP,), _F32),
        ],
    )(accp, g1, dis2d, dis1, b1.reshape(1, DH), W2,
      b2.reshape(1))


@functools.lru_cache(maxsize=None)
def _build_prop2_kernel(E, NP):
    EPT = E // _NT
    NPT = NP // _NSUB

    def body(g2_hbm, ei_hbm, out_hbm, g2b, srcb, dstb, acc2, outb, redb,
             slots):
        cid = lax.axis_index("c")
        sid = lax.axis_index("s")
        t = cid * _NSUB + sid
        pltpu.sync_copy(g2_hbm, g2b)
        zero = jnp.zeros((16,), _F32)

        def zb(i, c):
            base = i * 64
            for u in range(4):
                acc2[pl.ds(base + u * 16, 16)] = zero
            return c

        lax.fori_loop(0, NP // 64, zb, 0)
        pltpu.sync_copy(ei_hbm.at[0, pl.ds(t * EPT, EPT)], srcb)
        pltpu.sync_copy(ei_hbm.at[1, pl.ds(t * EPT, EPT)], dstb)

        def eb(i, c):
            base = i * 80
            for u in range(5):
                s = srcb[pl.ds(base + u * 16, 16)]
                d = dstb[pl.ds(base + u * 16, 16)]
                v = plsc.load_gather(g2b, [s])
                plsc.addupdate_scatter(acc2, [d], v)
            return c

        lax.fori_loop(0, EPT // 80, eb, 0)
        pltpu.sync_copy(acc2, slots.at[sid])
        plsc.subcore_barrier()
        for j in range(_NSUB):
            pltpu.sync_copy(slots.at[j, pl.ds(sid * NPT, NPT)], redb.at[j])

        def cb(ci, c):
            base = ci * 16
            tot = redb[0, pl.ds(base, 16)]
            for j in range(1, _NSUB):
                tot = tot + redb[j, pl.ds(base, 16)]
            outb[pl.ds(base, 16)] = tot
            return c

        lax.fori_loop(0, NPT // 16, cb, 0)
        pltpu.sync_copy(outb, out_hbm.at[cid, pl.ds(sid * NPT, NPT)])

    return pl.kernel(
        body,
        out_type=jax.ShapeDtypeStruct((2, NP), _F32),
        mesh=_sc_mesh(),
        scratch_types=[
            pltpu.VMEM((NP,), _F32),
            pltpu.VMEM((EPT,), jnp.int32),
            pltpu.VMEM((EPT,), jnp.int32),
            pltpu.VMEM((NP,), _F32),
            pltpu.VMEM((NPT,), _F32),
            pltpu.VMEM((_NSUB, NPT), _F32),
            pltpu.VMEM_SHARED((_NSUB, NP), _F32),
        ],
        **_SC_PARAMS,
    )


def _tc_final(p2, dis1, c2):
    NP = dis1.shape[0]

    def body(p_ref, dis_ref, c2_ref, out_ref):
        out_ref[...] = (dis_ref[...] * (p_ref[0] + p_ref[1])
                        + c2_ref[...])

    return pl.pallas_call(
        body,
        out_shape=jax.ShapeDtypeStruct((NP,), _F32),
    )(p2, dis1, c2)


def kernel(x, edge_index, W1, b1, W2, b2):
    N, _ = x.shape
    E = edge_index.shape[1]
    DH = W1.shape[1]
    NP = ((N + 16 * 128 - 1) // (16 * 128)) * (16 * 128)
    CH = 80
    NSLOT = 5

    p = _build_deg_kernel(E, NP)(edge_index)
    g1, dis2d, dis = _tc_prep(p, x, W1, NP)
    accp = _build_prop1_kernel(E, NP, DH, CH, NSLOT)(edge_index, g1)
    g2, c2 = _tc_mid(accp, g1, dis2d, dis, b1, W2, b2)
    p2 = _build_prop2_kernel(E, NP)(g2, edge_index)
    out = _tc_final(p2, dis, c2)
    return out[:N, None]


# speedup vs baseline: 1.0425x; 1.0162x over previous
import functools

import jax
import jax.numpy as jnp
from jax import lax
from jax.experimental import pallas as pl
from jax.experimental.pallas import tpu as pltpu
from jax.experimental.pallas import tpu_sc as plsc

_F32 = jnp.float32
_NT = 32
_NSUB = 16


def _sc_mesh():
    return plsc.VectorSubcoreMesh(core_axis_name="c", subcore_axis_name="s")


_SC_PARAMS = dict(
    compiler_params=pltpu.CompilerParams(
        needs_layout_passes=False, use_tc_tiling_on_sc=False),
)


@functools.lru_cache(maxsize=None)
def _build_deg_kernel(E, NP):
    EPT = E // _NT

    def body(ei_hbm, out_hbm, dstb, acc):
        cid = lax.axis_index("c")
        sid = lax.axis_index("s")
        t = cid * _NSUB + sid
        zero = jnp.zeros((16,), _F32)

        def zb(i, c):
            base = i * 64
            for u in range(4):
                acc[pl.ds(base + u * 16, 16)] = zero
            return c

        lax.fori_loop(0, NP // 64, zb, 0)
        pltpu.sync_copy(ei_hbm.at[1, pl.ds(t * EPT, EPT)], dstb)
        ones = jnp.ones((16,), _F32)

        def eb(i, c):
            base = i * 80
            for u in range(5):
                idx = dstb[pl.ds(base + u * 16, 16)]
                plsc.addupdate_scatter(acc, [idx], ones)
            return c

        lax.fori_loop(0, EPT // 80, eb, 0)
        pltpu.sync_copy(acc, out_hbm.at[t])

    return pl.kernel(
        body,
        out_type=jax.ShapeDtypeStruct((_NT, NP), _F32),
        mesh=_sc_mesh(),
        scratch_types=[
            pltpu.VMEM((EPT,), jnp.int32),
            pltpu.VMEM((NP,), _F32),
        ],
        **_SC_PARAMS,
    )


def _tc_matmul1(x, W1):
    N = x.shape[0]
    DH = W1.shape[1]

    def body(x_ref, w1_ref, h1_ref):
        h1_ref[...] = jnp.dot(x_ref[...], w1_ref[...],
                              preferred_element_type=_F32)

    return pl.pallas_call(
        body,
        out_shape=jax.ShapeDtypeStruct((N, DH), _F32),
    )(x, W1)


def _tc_prep(p, h1, NP):
    N, DH = h1.shape

    def body(p_ref, h1_ref, g1_ref, dis2_ref, dis1_ref):
        deg = jnp.sum(p_ref[...], axis=0) + 1.0
        dis = lax.rsqrt(deg)
        dis1_ref[...] = dis
        dis2_ref[...] = dis[:, None]
        g1_ref[pl.ds(0, N), :] = h1_ref[...] * dis[:N][:, None]

    return pl.pallas_call(
        body,
        out_shape=[
            jax.ShapeDtypeStruct((NP, DH), _F32),
            jax.ShapeDtypeStruct((NP, 1), _F32),
            jax.ShapeDtypeStruct((NP,), _F32),
        ],
    )(p, h1)


@functools.lru_cache(maxsize=None)
def _build_prop1_kernel(E, NP, DH, CH, NSLOT):
    EPT = E // _NT
    NIT = EPT // CH
    NPT = NP // _NSUB

    def body(ei_hbm, g1_hbm, out_hbm, srcb, dstb, rows, zbuf, sacc, *sems):
        gsem = sems[:NSLOT]
        ssem = sems[NSLOT:]
        cid = lax.axis_index("c")
        sid = lax.axis_index("s")
        t = cid * _NSUB + sid
        pltpu.sync_copy(ei_hbm.at[0, pl.ds(t * EPT, EPT)], srcb)
        pltpu.sync_copy(ei_hbm.at[1, pl.ds(t * EPT, EPT)], dstb)
        zero = jnp.zeros((16,), _F32)

        def zb(i, c):
            def zc(j, c2):
                zbuf[i, pl.ds(j * 16, 16)] = zero
                return c2
            lax.fori_loop(0, DH // 16, zc, 0)
            return c

        lax.fori_loop(0, 128, zb, 0)
        for r in range(NPT // 128):
            pltpu.sync_copy(zbuf, sacc.at[pl.ds(sid * NPT + r * 128, 128)])
        plsc.subcore_barrier()

        def start_g(i, s):
            pltpu.async_copy(g1_hbm.at[srcb.at[pl.ds(i * CH, CH)]],
                             rows.at[s], gsem[s])

        def wait_g(s):
            pltpu.make_async_copy(g1_hbm.at[srcb.at[pl.ds(0, CH)]],
                                  rows.at[s], gsem[s]).wait()

        def start_s(i, s):
            pltpu.async_copy(rows.at[s],
                             sacc.at[dstb.at[pl.ds(i * CH, CH)]],
                             ssem[s], add=True)

        def wait_s(s):
            pltpu.make_async_copy(rows.at[s],
                                  sacc.at[dstb.at[pl.ds(0, CH)]],
                                  ssem[s]).wait()

        for s in range(NSLOT):
            start_g(s, s)

        def eb(j, c):
            i0 = NSLOT * j
            for s in range(NSLOT):
                wait_g(s)
                start_s(i0 + s, s)

            @pl.when(j < NIT // NSLOT - 1)
            def _():
                for s in range(NSLOT):
                    wait_s(s)
                    start_g(i0 + NSLOT + s, s)

            return c

        lax.fori_loop(0, NIT // NSLOT, eb, 0)
        for s in range(NSLOT):
            wait_s(s)
        plsc.subcore_barrier()
        pltpu.sync_copy(sacc.at[pl.ds(sid * NPT, NPT)],
                        out_hbm.at[cid, pl.ds(sid * NPT, NPT)])

    return pl.kernel(
        body,
        out_type=jax.ShapeDtypeStruct((2, NP, DH), _F32),
        mesh=_sc_mesh(),
        scratch_types=(
            [pltpu.VMEM((EPT,), jnp.int32),
             pltpu.VMEM((EPT,), jnp.int32),
             pltpu.VMEM((NSLOT, CH, DH), _F32),
             pltpu.VMEM((128, DH), _F32),
             pltpu.VMEM_SHARED((NP, DH), _F32)]
            + [pltpu.SemaphoreType.DMA] * (2 * NSLOT)
        ),
        **_SC_PARAMS,
    )


def _tc_mid(accp, g1, dis2d, dis1, b1, W2, b2):
    NP, DH = g1.shape

    def body(a_ref, g1_ref, dis2_ref, dis1_ref, b1_ref, w2_ref, b2_ref,
             g2_ref, c2_ref):
        acc = a_ref[0] + a_ref[1] + g1_ref[...]
        out1 = acc * dis2_ref[...] + b1_ref[...]
        h = jnp.maximum(out1, 0.0)
        h2 = jnp.dot(h, w2_ref[...],
                     preferred_element_type=_F32)[:, 0]
        dis = dis1_ref[...]
        g2 = dis * h2
        g2_ref[...] = g2
        c2_ref[...] = dis * g2 + b2_ref[...]

    return pl.pallas_call(
        body,
        out_shape=[
            jax.ShapeDtypeStruct((NP,), _F32),
            jax.ShapeDtypeStruct((NP,), _F32),
        ],
    )(accp, g1, dis2d, dis1, b1.reshape(1, DH), W2,
      b2.reshape(1))


@functools.lru_cache(maxsize=None)
def _build_prop2_kernel(E, NP):
    EPT = E // _NT
    NPT = NP // _NSUB

    def body(g2_hbm, ei_hbm, out_hbm, g2b, srcb, dstb, acc2, outb, redb,
             slots):
        cid = lax.axis_index("c")
        sid = lax.axis_index("s")
        t = cid * _NSUB + sid
        pltpu.sync_copy(g2_hbm, g2b)
        zero = jnp.zeros((16,), _F32)

        def zb(i, c):
            base = i * 64
            for u in range(4):
                acc2[pl.ds(base + u * 16, 16)] = zero
            return c

        lax.fori_loop(0, NP // 64, zb, 0)
        pltpu.sync_copy(ei_hbm.at[0, pl.ds(t * EPT, EPT)], srcb)
        pltpu.sync_copy(ei_hbm.at[1, pl.ds(t * EPT, EPT)], dstb)

        def eb(i, c):
            base = i * 80
            for u in range(5):
                s = srcb[pl.ds(base + u * 16, 16)]
                d = dstb[pl.ds(base + u * 16, 16)]
                v = plsc.load_gather(g2b, [s])
                plsc.addupdate_scatter(acc2, [d], v)
            return c

        lax.fori_loop(0, EPT // 80, eb, 0)
        pltpu.sync_copy(acc2, slots.at[sid])
        plsc.subcore_barrier()
        for j in range(_NSUB):
            pltpu.sync_copy(slots.at[j, pl.ds(sid * NPT, NPT)], redb.at[j])

        def cb(ci, c):
            base = ci * 16
            tot = redb[0, pl.ds(base, 16)]
            for j in range(1, _NSUB):
                tot = tot + redb[j, pl.ds(base, 16)]
            outb[pl.ds(base, 16)] = tot
            return c

        lax.fori_loop(0, NPT // 16, cb, 0)
        pltpu.sync_copy(outb, out_hbm.at[cid, pl.ds(sid * NPT, NPT)])

    return pl.kernel(
        body,
        out_type=jax.ShapeDtypeStruct((2, NP), _F32),
        mesh=_sc_mesh(),
        scratch_types=[
            pltpu.VMEM((NP,), _F32),
            pltpu.VMEM((EPT,), jnp.int32),
            pltpu.VMEM((EPT,), jnp.int32),
            pltpu.VMEM((NP,), _F32),
            pltpu.VMEM((NPT,), _F32),
            pltpu.VMEM((_NSUB, NPT), _F32),
            pltpu.VMEM_SHARED((_NSUB, NP), _F32),
        ],
        **_SC_PARAMS,
    )


def _tc_final(p2, dis1, c2):
    NP = dis1.shape[0]

    def body(p_ref, dis_ref, c2_ref, out_ref):
        out_ref[...] = (dis_ref[...] * (p_ref[0] + p_ref[1])
                        + c2_ref[...])

    return pl.pallas_call(
        body,
        out_shape=jax.ShapeDtypeStruct((NP,), _F32),
    )(p2, dis1, c2)


def kernel(x, edge_index, W1, b1, W2, b2):
    N, _ = x.shape
    E = edge_index.shape[1]
    DH = W1.shape[1]
    NP = ((N + 16 * 128 - 1) // (16 * 128)) * (16 * 128)
    CH = 80
    NSLOT = 5

    h1 = _tc_matmul1(x, W1)
    p = _build_deg_kernel(E, NP)(edge_index)
    g1, dis2d, dis = _tc_prep(p, h1, NP)
    accp = _build_prop1_kernel(E, NP, DH, CH, NSLOT)(edge_index, g1)
    g2, c2 = _tc_mid(accp, g1, dis2d, dis, b1, W2, b2)
    p2 = _build_prop2_kernel(E, NP)(g2, edge_index)
    out = _tc_final(p2, dis, c2)
    return out[:N, None]


# speedup vs baseline: 1.0530x; 1.0101x over previous
import functools

import jax
import jax.numpy as jnp
from jax import lax
from jax.experimental import pallas as pl
from jax.experimental.pallas import tpu as pltpu
from jax.experimental.pallas import tpu_sc as plsc

_F32 = jnp.float32
_NT = 32
_NSUB = 16


def _sc_mesh():
    return plsc.VectorSubcoreMesh(core_axis_name="c", subcore_axis_name="s")


_SC_PARAMS = dict(
    compiler_params=pltpu.CompilerParams(
        needs_layout_passes=False, use_tc_tiling_on_sc=False),
)


@functools.lru_cache(maxsize=None)
def _build_deg_kernel(E, NP):
    EPT = E // _NT

    def body(ei_hbm, out_hbm, dstb, acc):
        cid = lax.axis_index("c")
        sid = lax.axis_index("s")
        t = cid * _NSUB + sid
        zero = jnp.zeros((16,), _F32)

        def zb(i, c):
            base = i * 64
            for u in range(4):
                acc[pl.ds(base + u * 16, 16)] = zero
            return c

        lax.fori_loop(0, NP // 64, zb, 0)
        pltpu.sync_copy(ei_hbm.at[1, pl.ds(t * EPT, EPT)], dstb)
        ones = jnp.ones((16,), _F32)

        def eb(i, c):
            base = i * 80
            for u in range(5):
                idx = dstb[pl.ds(base + u * 16, 16)]
                plsc.addupdate_scatter(acc, [idx], ones)
            return c

        lax.fori_loop(0, EPT // 80, eb, 0)
        pltpu.sync_copy(acc, out_hbm.at[t])

    return pl.kernel(
        body,
        out_type=jax.ShapeDtypeStruct((_NT, NP), _F32),
        mesh=_sc_mesh(),
        scratch_types=[
            pltpu.VMEM((EPT,), jnp.int32),
            pltpu.VMEM((NP,), _F32),
        ],
        **_SC_PARAMS,
    )


def _tc_matmul1(x, W1):
    N = x.shape[0]
    DH = W1.shape[1]

    def body(x_ref, w1_ref, h1_ref):
        h1_ref[...] = jnp.dot(x_ref[...], w1_ref[...],
                              preferred_element_type=_F32)

    return pl.pallas_call(
        body,
        out_shape=jax.ShapeDtypeStruct((N, DH), _F32),
    )(x, W1)


def _tc_prep(p, h1, NP):
    N, DH = h1.shape
    G = 5
    BL = NP // G

    def body(p_ref, h1_ref, g1_ref, dis2_ref, dis1_ref):
        deg = jnp.sum(p_ref[...], axis=0) + 1.0
        dis = lax.rsqrt(deg)
        dis1_ref[...] = dis
        dis2_ref[...] = dis[:, None]
        g1_ref[...] = h1_ref[...] * dis[:, None]

    return pl.pallas_call(
        body,
        grid=(G,),
        in_specs=[
            pl.BlockSpec((32, BL), lambda i: (0, i)),
            pl.BlockSpec((BL, DH), lambda i: (i, 0)),
        ],
        out_specs=[
            pl.BlockSpec((BL, DH), lambda i: (i, 0)),
            pl.BlockSpec((BL, 1), lambda i: (i, 0)),
            pl.BlockSpec((BL,), lambda i: (i,)),
        ],
        out_shape=[
            jax.ShapeDtypeStruct((NP, DH), _F32),
            jax.ShapeDtypeStruct((NP, 1), _F32),
            jax.ShapeDtypeStruct((NP,), _F32),
        ],
    )(p, h1)


@functools.lru_cache(maxsize=None)
def _build_prop1_kernel(E, NP, DH, CH, NSLOT):
    EPT = E // _NT
    NIT = EPT // CH
    NPT = NP // _NSUB

    def body(ei_hbm, g1_hbm, out_hbm, srcb, dstb, rows, zbuf, sacc, *sems):
        gsem = sems[:NSLOT]
        ssem = sems[NSLOT:]
        cid = lax.axis_index("c")
        sid = lax.axis_index("s")
        t = cid * _NSUB + sid
        pltpu.sync_copy(ei_hbm.at[0, pl.ds(t * EPT, EPT)], srcb)
        pltpu.sync_copy(ei_hbm.at[1, pl.ds(t * EPT, EPT)], dstb)
        zero = jnp.zeros((16,), _F32)

        def zb(i, c):
            def zc(j, c2):
                zbuf[i, pl.ds(j * 16, 16)] = zero
                return c2
            lax.fori_loop(0, DH // 16, zc, 0)
            return c

        lax.fori_loop(0, 128, zb, 0)
        for r in range(NPT // 128):
            pltpu.sync_copy(zbuf, sacc.at[pl.ds(sid * NPT + r * 128, 128)])
        plsc.subcore_barrier()

        def start_g(i, s):
            pltpu.async_copy(g1_hbm.at[srcb.at[pl.ds(i * CH, CH)]],
                             rows.at[s], gsem[s])

        def wait_g(s):
            pltpu.make_async_copy(g1_hbm.at[srcb.at[pl.ds(0, CH)]],
                                  rows.at[s], gsem[s]).wait()

        def start_s(i, s):
            pltpu.async_copy(rows.at[s],
                             sacc.at[dstb.at[pl.ds(i * CH, CH)]],
                             ssem[s], add=True)

        def wait_s(s):
            pltpu.make_async_copy(rows.at[s],
                                  sacc.at[dstb.at[pl.ds(0, CH)]],
                                  ssem[s]).wait()

        for s in range(NSLOT):
            start_g(s, s)

        def eb(j, c):
            i0 = NSLOT * j
            for s in range(NSLOT):
                wait_g(s)
                start_s(i0 + s, s)

            @pl.when(j < NIT // NSLOT - 1)
            def _():
                for s in range(NSLOT):
                    wait_s(s)
                    start_g(i0 + NSLOT + s, s)

            return c

        lax.fori_loop(0, NIT // NSLOT, eb, 0)
        for s in range(NSLOT):
            wait_s(s)
        plsc.subcore_barrier()
        pltpu.sync_copy(sacc.at[pl.ds(sid * NPT, NPT)],
                        out_hbm.at[cid, pl.ds(sid * NPT, NPT)])

    return pl.kernel(
        body,
        out_type=jax.ShapeDtypeStruct((2, NP, DH), _F32),
        mesh=_sc_mesh(),
        scratch_types=(
            [pltpu.VMEM((EPT,), jnp.int32),
             pltpu.VMEM((EPT,), jnp.int32),
             pltpu.VMEM((NSLOT, CH, DH), _F32),
             pltpu.VMEM((128, DH), _F32),
             pltpu.VMEM_SHARED((NP, DH), _F32)]
            + [pltpu.SemaphoreType.DMA] * (2 * NSLOT)
        ),
        **_SC_PARAMS,
    )


def _tc_mid(accp, g1, dis2d, dis1, b1, W2, b2):
    NP, DH = g1.shape

    G = 5
    BL = NP // G

    def body(a_ref, g1_ref, dis2_ref, dis1_ref, b1_ref, w2_ref, b2_ref,
             g2_ref, c2_ref):
        acc = a_ref[0] + a_ref[1] + g1_ref[...]
        out1 = acc * dis2_ref[...] + b1_ref[...]
        h = jnp.maximum(out1, 0.0)
        h2 = jnp.dot(h, w2_ref[...],
                     preferred_element_type=_F32)[:, 0]
        dis = dis1_ref[...]
        g2 = dis * h2
        g2_ref[...] = g2
        c2_ref[...] = dis * g2 + b2_ref[...]

    return pl.pallas_call(
        body,
        grid=(G,),
        in_specs=[
            pl.BlockSpec((2, BL, DH), lambda i: (0, i, 0)),
            pl.BlockSpec((BL, DH), lambda i: (i, 0)),
            pl.BlockSpec((BL, 1), lambda i: (i, 0)),
            pl.BlockSpec((BL,), lambda i: (i,)),
            pl.BlockSpec((1, DH), lambda i: (0, 0)),
            pl.BlockSpec((DH, 1), lambda i: (0, 0)),
            pl.BlockSpec((1,), lambda i: (0,)),
        ],
        out_specs=[
            pl.BlockSpec((BL,), lambda i: (i,)),
            pl.BlockSpec((BL,), lambda i: (i,)),
        ],
        out_shape=[
            jax.ShapeDtypeStruct((NP,), _F32),
            jax.ShapeDtypeStruct((NP,), _F32),
        ],
    )(accp, g1, dis2d, dis1, b1.reshape(1, DH), W2,
      b2.reshape(1))


@functools.lru_cache(maxsize=None)
def _build_prop2_kernel(E, NP):
    EPT = E // _NT
    NPT = NP // _NSUB

    def body(g2_hbm, ei_hbm, out_hbm, g2b, srcb, dstb, acc2, outb, redb,
             slots):
        cid = lax.axis_index("c")
        sid = lax.axis_index("s")
        t = cid * _NSUB + sid
        pltpu.sync_copy(g2_hbm, g2b)
        zero = jnp.zeros((16,), _F32)

        def zb(i, c):
            base = i * 64
            for u in range(4):
                acc2[pl.ds(base + u * 16, 16)] = zero
            return c

        lax.fori_loop(0, NP // 64, zb, 0)
        pltpu.sync_copy(ei_hbm.at[0, pl.ds(t * EPT, EPT)], srcb)
        pltpu.sync_copy(ei_hbm.at[1, pl.ds(t * EPT, EPT)], dstb)

        def eb(i, c):
            base = i * 80
            for u in range(5):
                s = srcb[pl.ds(base + u * 16, 16)]
                d = dstb[pl.ds(base + u * 16, 16)]
                v = plsc.load_gather(g2b, [s])
                plsc.addupdate_scatter(acc2, [d], v)
            return c

        lax.fori_loop(0, EPT // 80, eb, 0)
        pltpu.sync_copy(acc2, slots.at[sid])
        plsc.subcore_barrier()
        for j in range(_NSUB):
            pltpu.sync_copy(slots.at[j, pl.ds(sid * NPT, NPT)], redb.at[j])

        def cb(ci, c):
            base = ci * 16
            tot = redb[0, pl.ds(base, 16)]
            for j in range(1, _NSUB):
                tot = tot + redb[j, pl.ds(base, 16)]
            outb[pl.ds(base, 16)] = tot
            return c

        lax.fori_loop(0, NPT // 16, cb, 0)
        pltpu.sync_copy(outb, out_hbm.at[cid, pl.ds(sid * NPT, NPT)])

    return pl.kernel(
        body,
        out_type=jax.ShapeDtypeStruct((2, NP), _F32),
        mesh=_sc_mesh(),
        scratch_types=[
            pltpu.VMEM((NP,), _F32),
            pltpu.VMEM((EPT,), jnp.int32),
            pltpu.VMEM((EPT,), jnp.int32),
            pltpu.VMEM((NP,), _F32),
            pltpu.VMEM((NPT,), _F32),
            pltpu.VMEM((_NSUB, NPT), _F32),
            pltpu.VMEM_SHARED((_NSUB, NP), _F32),
        ],
        **_SC_PARAMS,
    )


def _tc_final(p2, dis1, c2):
    NP = dis1.shape[0]

    def body(p_ref, dis_ref, c2_ref, out_ref):
        out_ref[...] = (dis_ref[...] * (p_ref[0] + p_ref[1])
                        + c2_ref[...])

    return pl.pallas_call(
        body,
        out_shape=jax.ShapeDtypeStruct((NP,), _F32),
    )(p2, dis1, c2)


def kernel(x, edge_index, W1, b1, W2, b2):
    N, _ = x.shape
    E = edge_index.shape[1]
    DH = W1.shape[1]
    NP = ((N + 16 * 128 - 1) // (16 * 128)) * (16 * 128)
    CH = 80
    NSLOT = 5

    h1 = _tc_matmul1(x, W1)
    p = _build_deg_kernel(E, NP)(edge_index)
    g1, dis2d, dis = _tc_prep(p, h1, NP)
    accp = _build_prop1_kernel(E, NP, DH, CH, NSLOT)(edge_index, g1)
    g2, c2 = _tc_mid(accp, g1, dis2d, dis, b1, W2, b2)
    p2 = _build_prop2_kernel(E, NP)(g2, edge_index)
    out = _tc_final(p2, dis, c2)
    return out[:N, None]
